# baseline (speedup 1.0000x reference)
import jax
import jax.numpy as jnp
from jax.experimental import pallas as pl

NCLUSTER = 512
PTS = 512


def _body(x_ref, ct_ref, out_ref):
    x0 = x_ref[:, 0:1]
    x1 = x_ref[:, 1:2]
    x2 = x_ref[:, 2:3]
    c0 = ct_ref[0:1, :]
    c1 = ct_ref[1:2, :]
    c2 = ct_ref[2:3, :]
    d = (x0 - c0) ** 2 + (x1 - c1) ** 2 + (x2 - c2) ** 2
    a = jnp.argmin(d, axis=-1).astype(jnp.int32)
    out_ref[0, 0, :] = a


def kernel(x, C):
    bs, c, h, w = x.shape
    n = bs * h * w
    xt = x.reshape(bs, c, h * w).transpose(0, 2, 1).reshape(n, c)
    ct = C.T
    grid = n // PTS
    out = pl.pallas_call(
        _body,
        grid=(grid,),
        in_specs=[
            pl.BlockSpec((PTS, c), lambda i: (i, 0)),
            pl.BlockSpec((c, NCLUSTER), lambda i: (0, 0)),
        ],
        out_specs=pl.BlockSpec((1, 1, PTS), lambda i: (i, 0, 0)),
        out_shape=jax.ShapeDtypeStruct((grid, 1, PTS), jnp.int32),
    )(xt, ct)
    return out.reshape(bs, h * w)
